# baseline (speedup 1.0000x reference)
import functools
import jax
import jax.numpy as jnp
from jax import lax
from jax.experimental import pallas as pl
from jax.experimental.pallas import tpu as pltpu
from jax.experimental.pallas import tpu_sc as plsc

BATCH, SEQ, DIM = 4096, 200, 64
K = SEQ * DIM
NW = 32
NCHUNK = 3200
CPW = NCHUNK // NW
KPW = K // NW
NBUF = 5
NGRP = CPW // NBUF


def _sc_body(x_hbm, pos_hbm, out_hbm, pv, bufs, *sems):
    gsem = sems[:NBUF]
    ssem = sems[NBUF:]
    wid = lax.axis_index("s") * 2 + lax.axis_index("c")
    base_m = wid * CPW
    base_k = wid * KPW

    pltpu.sync_copy(pos_hbm.at[pl.ds(base_k, KPW)], pv)

    def gather(m, p):
        return pltpu.make_async_copy(x_hbm.at[m], bufs.at[p], gsem[p])

    def scatter(m, p):
        return pltpu.make_async_copy(bufs.at[p], out_hbm.at[m], ssem[p])

    for p in range(NBUF):
        gather(base_m + p, p).start()

    def gbody(g, carry):
        for p in range(NBUF):
            m = base_m + g * NBUF + p
            gather(m, p).wait()
            k0 = 8 * lax.shift_right_logical(m, 1) - base_k
            buf = bufs.at[p]
            for i in range(8):
                idx = jnp.full((16,), k0 + i, dtype=jnp.int32)
                splat = plsc.load_gather(pv, [idx])

                def cbody(c, carry2, buf=buf, i=i, splat=splat):
                    for t in range(8):
                        sl = pl.ds(t * 16, 16)
                        buf[c, i, sl] = buf[c, i, sl] + splat
                    return carry2

                lax.fori_loop(0, 16, cbody, 0)
            scatter(m, p).start()
        for p in range(NBUF):
            m = base_m + g * NBUF + p
            scatter(m, p).wait()

            @pl.when(g < NGRP - 1)
            def _(p=p, g=g):
                gather(base_m + (g + 1) * NBUF + p, p).start()

        return carry

    lax.fori_loop(0, NGRP, gbody, 0)


def kernel(x, pos_table):
    b, s, d = x.shape
    k = s * d
    xt = jnp.transpose(x, (1, 2, 0)).reshape(k, b)
    x4 = xt.reshape(k // 8, 8, b // 128, 128).transpose(0, 2, 1, 3)
    x5 = x4.reshape(NCHUNK, 16, 8, 128)
    posf = pos_table.reshape(k)

    mesh = plsc.VectorSubcoreMesh(core_axis_name="c", subcore_axis_name="s")
    f = functools.partial(
        pl.kernel,
        mesh=mesh,
        out_type=jax.ShapeDtypeStruct((NCHUNK, 16, 8, 128), jnp.float32),
        scratch_types=[
            pltpu.VMEM((KPW,), jnp.float32),
            pltpu.VMEM((NBUF, 16, 8, 128), jnp.float32),
        ]
        + [pltpu.SemaphoreType.DMA] * (2 * NBUF),
        compiler_params=pltpu.CompilerParams(needs_layout_passes=False),
    )(_sc_body)
    out5 = f(x5, posf)

    out_xt = out5.reshape(k // 8, b // 128, 8, 128).transpose(0, 2, 1, 3).reshape(k, b)
    return jnp.transpose(out_xt.reshape(s, d, b), (2, 0, 1))
